# Initial kernel scaffold; baseline (speedup 1.0000x reference)
import jax
import jax.numpy as jnp
from jax.experimental import pallas as pl


def kernel(edge_src, edge_dst, cell_feature, gene_feature, W1, b1, gamma, beta, layer_weights):
    raise NotImplementedError("write your pallas kernel here")


# speedup vs baseline: 2.8081x; 2.8081x over previous
import functools

import jax
import jax.numpy as jnp
from jax import lax
from jax.experimental import pallas as pl
from jax.experimental.pallas import tpu as pltpu
from jax.experimental.pallas import tpu_sc as plsc

_E = 600000
_ER = 4864
_EP = _ER * 128
_D = 128
_NU, _NI = 50000, 10000
_PU, _PI = 51200, 10240
_CPT = _ER // 16
_DCPT = _ER // 32
_RTU = _PU // (16 * 128)
_RTI = _PI // (16 * 128)

_F32 = jnp.float32
_I32 = jnp.int32

_SC_PARAMS = pltpu.CompilerParams(needs_layout_passes=False,
                                  use_tc_tiling_on_sc=False)


@functools.lru_cache(maxsize=None)
def _mesh():
    return plsc.VectorSubcoreMesh(core_axis_name="c", subcore_axis_name="s")


def _splat(ref, j):
    return plsc.load_gather(ref, [jnp.full((16,), j, _I32)])


def _fill(ref, val, rows, width):
    v = jnp.full((16,), val, _F32)
    if rows is None:
        for b in range(width // 16):
            ref[pl.ds(16 * b, 16)] = v
        return

    def body(r, _):
        for b in range(width // 16):
            ref[r, pl.ds(16 * b, 16)] = v
        return _

    lax.fori_loop(0, rows, body, None)


def _k0a_body(src2d, dst2d, degu, degi, acc, idxb, onesv, tmpv, ssem):
    sid = lax.axis_index("s")
    cid = lax.axis_index("c")

    def side(idx2d, deg_out, rchunks):
        t = sid
        _fill(tmpv, 0.0, None, 128)
        zd = [pltpu.async_copy(
                  tmpv, acc.at[pl.ds((t * rchunks + r) * 128, 128)], ssem)
              for r in range(rchunks)]
        for zz in zd:
            zz.wait()
        plsc.subcore_barrier()

        pltpu.sync_copy(idx2d.at[pl.ds(t * _CPT, _CPT)], idxb)
        _fill(onesv, 1.0, None, 128)

        def sb(j8, _):
            sd = [pltpu.async_copy(onesv, acc.at[idxb.at[j8 * 8 + jj]],
                                   ssem, add=True)
                  for jj in range(8)]
            for ss in sd:
                ss.wait()
            return _

        lax.fori_loop(0, _CPT // 8, sb, None)
        plsc.subcore_barrier()

        def wb(r, _):
            R = (t * rchunks + r) * 128
            pltpu.sync_copy(acc.at[pl.ds(R, 128)], tmpv)
            pltpu.sync_copy(tmpv, deg_out.at[pl.ds(R, 128)])
            return _

        lax.fori_loop(0, rchunks, wb, None)

    @pl.when(cid == 0)
    def _():
        side(src2d, degu, _RTU)

    @pl.when(cid == 1)
    def _():
        side(dst2d, degi, _RTI)


def _k0a(src2d, dst2d):
    out_type = (
        jax.ShapeDtypeStruct((_PU,), _F32),
        jax.ShapeDtypeStruct((_PI,), _F32),
    )
    scratch = [
        pltpu.VMEM_SHARED((_PU,), _F32),
        pltpu.VMEM((_CPT, 128), _I32),
        pltpu.VMEM((128,), _F32),
        pltpu.VMEM((128,), _F32),
        pltpu.SemaphoreType.DMA,
    ]
    f = pl.kernel(_k0a_body, out_type=out_type, mesh=_mesh(),
                  scratch_types=scratch, compiler_params=_SC_PARAMS)
    return f(src2d, dst2d)


def _tc_c_body(d, o):
    dv = d[...]
    o[...] = jnp.where(dv > 0.0, lax.rsqrt(jnp.maximum(dv, 1.0)), 0.0)


def _tc_c(deg):
    n = deg.shape[0]
    d2 = deg.reshape(n // 128, 128)
    blk = pl.BlockSpec((n // 128, 128), lambda: (0, 0))
    c2 = pl.pallas_call(
        _tc_c_body,
        in_specs=[blk],
        out_specs=blk,
        out_shape=jax.ShapeDtypeStruct((n // 128, 128), _F32),
    )(d2)
    return c2.reshape(n)


def _k0c_body(u0p, i0p, cuv, civ, up0, up1, up2, up3, ip0, ip1, ip2, ip3,
              cbuf, fbuf, g0, g1, g2, g3):
    sid = lax.axis_index("s")
    cid = lax.axis_index("c")
    gbufs = [g0, g1, g2, g3]

    def side(feat, c_in, gouts, rchunks):
        t = sid

        def wb(r, _):
            R = (t * rchunks + r) * 128
            pltpu.sync_copy(c_in.at[pl.ds(R, 128)], cbuf)
            pltpu.sync_copy(feat.at[pl.ds(R, 128)], fbuf)

            def rb(r2, _):
                cs = _splat(cbuf, r2)
                for b in range(8):
                    v = fbuf[r2, pl.ds(16 * b, 16)] * cs
                    gbufs[b // 2][r2, pl.ds(16 * (b % 2), 16)] = v
                return _

            lax.fori_loop(0, 128, rb, None)
            for gb, go in zip(gbufs, gouts):
                pltpu.sync_copy(gb, go.at[pl.ds(R, 128)])
            return _

        lax.fori_loop(0, rchunks, wb, None)

    @pl.when(cid == 0)
    def _():
        side(u0p, cuv, [up0, up1, up2, up3], _RTU)

    @pl.when(cid == 1)
    def _():
        side(i0p, civ, [ip0, ip1, ip2, ip3], _RTI)


def _k0c(u0p, i0p, cu, ci):
    out_type = tuple([jax.ShapeDtypeStruct((_PU, 32), _F32)] * 4
                     + [jax.ShapeDtypeStruct((_PI, 32), _F32)] * 4)
    scratch = [
        pltpu.VMEM((128,), _F32),
        pltpu.VMEM((128, 128), _F32),
        pltpu.VMEM((128, 32), _F32),
        pltpu.VMEM((128, 32), _F32),
        pltpu.VMEM((128, 32), _F32),
        pltpu.VMEM((128, 32), _F32),
    ]
    f = pl.kernel(_k0c_body, out_type=out_type, mesh=_mesh(),
                  scratch_types=scratch, compiler_params=_SC_PARAMS)
    return f(u0p, i0p, cu, ci)


def _klayer_body(nrows, rchunks, swap, primed,
                 src2d, dst2d, t0, t1, t2, t3, cvec, *refs):
    nat = refs[:4]
    pr = refs[4:8] if primed else (None,) * 4
    (acc, sidx, didx, rows4, cbuf, abuf, prb,
     gsem, ssem, wsem) = refs[8:] if primed else refs[4:]
    tbls = [t0, t1, t2, t3]
    sid = lax.axis_index("s")
    cid = lax.axis_index("c")
    t = sid

    gsrc, xsrc = (dst2d, src2d) if swap else (src2d, dst2d)

    def gpass(tbl, natout, prout):
        _fill(abuf, 0.0, 128, 32)
        zd = [pltpu.async_copy(
                  abuf, acc.at[pl.ds((t * rchunks + r) * 128, 128)], wsem)
              for r in range(rchunks)]
        for zz in zd:
            zz.wait()
        plsc.subcore_barrier()

        def sb(j8, _):
            base = t * _CPT + j8 * 8
            pltpu.sync_copy(gsrc.at[pl.ds(base, 8)], sidx)
            pltpu.sync_copy(xsrc.at[pl.ds(base, 8)], didx)
            sd = []
            for w in range(2):
                if w == 1:
                    for ss in sd:
                        ss.wait()
                    sd = []
                gd = [pltpu.async_copy(tbl.at[sidx.at[4 * w + jj]],
                                       rows4.at[jj], gsem)
                      for jj in range(4)]
                for jj in range(4):
                    gd[jj].wait()
                    sd.append(pltpu.async_copy(
                        rows4.at[jj], acc.at[didx.at[4 * w + jj]],
                        ssem, add=True))
            for ss in sd:
                ss.wait()
            return _

        lax.fori_loop(0, _CPT // 8, sb, None)
        plsc.subcore_barrier()

        wd = []
        for r in range(rchunks):
            R = (t * rchunks + r) * 128
            for d in wd:
                d.wait()
            wd = []
            pltpu.sync_copy(acc.at[pl.ds(R, 128)], abuf)
            pltpu.sync_copy(cvec.at[pl.ds(R, 128)], cbuf)

            def rb(r2, _):
                cs = _splat(cbuf, r2)
                cs2 = cs * cs
                for b in range(2):
                    v = abuf[r2, pl.ds(16 * b, 16)]
                    if primed:
                        prb[r2, pl.ds(16 * b, 16)] = v * cs2
                    abuf[r2, pl.ds(16 * b, 16)] = v * cs
                return _

            lax.fori_loop(0, 128, rb, None)
            wd = [pltpu.async_copy(abuf, natout.at[pl.ds(R, 128)], wsem)]
            if primed:
                wd.append(pltpu.async_copy(prb, prout.at[pl.ds(R, 128)],
                                           wsem))
        for d in wd:
            d.wait()
        plsc.subcore_barrier()

    for gp in range(2):
        @pl.when(cid == 0)
        def _(gp=gp):
            gpass(tbls[gp], nat[gp], pr[gp])

        @pl.when(cid == 1)
        def _(gp=gp):
            gpass(tbls[2 + gp], nat[2 + gp], pr[2 + gp])


def _klayer(swap, primed, src2d, dst2d, tbls, cvec):
    nrows = _PU if swap else _PI
    rchunks = _RTU if swap else _RTI
    nout = 8 if primed else 4
    out_type = tuple([jax.ShapeDtypeStruct((nrows, 32), _F32)] * nout)
    scratch = [
        pltpu.VMEM_SHARED((nrows, 32), _F32),
        pltpu.VMEM((8, 128), _I32),
        pltpu.VMEM((8, 128), _I32),
        pltpu.VMEM((4, 128, 32), _F32),
        pltpu.VMEM((128,), _F32),
        pltpu.VMEM((128, 32), _F32),
        pltpu.VMEM((128, 32), _F32),
        pltpu.SemaphoreType.DMA,
        pltpu.SemaphoreType.DMA,
        pltpu.SemaphoreType.DMA,
    ]
    body = functools.partial(_klayer_body, nrows, rchunks, swap, primed)
    f = pl.kernel(body, out_type=out_type, mesh=_mesh(),
                  scratch_types=scratch, compiler_params=_SC_PARAMS)
    outs = f(src2d, dst2d, tbls[0], tbls[1], tbls[2], tbls[3], cvec)
    return outs[:4], outs[4:]


def _kdec_body(src2d, dst2d, uemb, ifin, out2d, sidx, didx, ur2, ir2, sbuf,
               gsem):
    sid = lax.axis_index("s")
    cid = lax.axis_index("c")
    wid = sid * 2 + cid

    pltpu.sync_copy(src2d.at[pl.ds(wid * _DCPT, _DCPT)], sidx)
    pltpu.sync_copy(dst2d.at[pl.ds(wid * _DCPT, _DCPT)], didx)
    lane0 = lax.iota(_I32, 16) == 0

    def fire(k, bb):
        pltpu.async_copy(uemb.at[sidx.at[k]], ur2.at[bb], gsem)
        pltpu.async_copy(ifin.at[didx.at[k]], ir2.at[bb], gsem)

    def drain(bb):
        pltpu.make_async_copy(uemb.at[sidx.at[0]], ur2.at[bb], gsem).wait()
        pltpu.make_async_copy(ifin.at[didx.at[0]], ir2.at[bb], gsem).wait()

    def compute(k, bb):
        ur = ur2.at[bb]
        ir = ir2.at[bb]

        def eb(e, _):
            a = ur[e, pl.ds(0, 16)] * ir[e, pl.ds(0, 16)]
            for b in range(1, 8):
                a = a + ur[e, pl.ds(16 * b, 16)] * ir[e, pl.ds(16 * b, 16)]
            s = jnp.sum(a)
            plsc.store_scatter(sbuf, [jnp.full((16,), e, _I32)],
                               jnp.full((16,), s, _F32), mask=lane0)
            return _

        lax.fori_loop(0, 128, eb, None)
        for b in range(8):
            x = sbuf[pl.ds(16 * b, 16)]
            sbuf[pl.ds(16 * b, 16)] = 1.0 / (1.0 + jnp.exp(-x))
        pltpu.sync_copy(sbuf, out2d.at[wid * _DCPT + k])

    npairs = _DCPT // 2
    fire(0, 0)

    def db(j2, _):
        k0 = 2 * j2
        fire(k0 + 1, 1)
        drain(0)
        compute(k0, 0)

        @pl.when(j2 + 1 < npairs)
        def _():
            fire(k0 + 2, 0)

        drain(1)
        compute(k0 + 1, 1)
        return _

    lax.fori_loop(0, npairs, db, None)


def _kdec(src2d, dst2d, uemb, ifin):
    scratch = [
        pltpu.VMEM((_DCPT, 128), _I32),
        pltpu.VMEM((_DCPT, 128), _I32),
        pltpu.VMEM((2, 128, 128), _F32),
        pltpu.VMEM((2, 128, 128), _F32),
        pltpu.VMEM((128,), _F32),
        pltpu.SemaphoreType.DMA,
    ]
    f = pl.kernel(_kdec_body,
                  out_type=jax.ShapeDtypeStruct((_ER, 128), _F32),
                  mesh=_mesh(), scratch_types=scratch,
                  compiler_params=_SC_PARAMS)
    return f(src2d, dst2d, uemb, ifin)


def _tc_ifinal_body(i0, i1, i2, w, o):
    o[...] = w[0] * i0[...] + w[1] * i1[...] + w[2] * i2[...]


def _tc_ifinal(i0p, i1, i2, lw):
    blk = lambda: pl.BlockSpec((128, _D), lambda r: (r, 0))
    return pl.pallas_call(
        _tc_ifinal_body,
        grid=(_PI // 128,),
        in_specs=[blk(), blk(), blk(),
                  pl.BlockSpec(memory_space=pltpu.SMEM)],
        out_specs=blk(),
        out_shape=jax.ShapeDtypeStruct((_PI, _D), _F32),
    )(i0p, i1, i2, lw)


def _tc_mm_body(u0, u1, u2, w1, b1, w, h, stats, acc):
    r = pl.program_id(0)

    @pl.when(r == 0)
    def _():
        acc[...] = jnp.zeros_like(acc)

    ufin = w[0] * u0[...] + w[1] * u1[...] + w[2] * u2[...]
    hb = jnp.dot(ufin, w1[0:128, :], preferred_element_type=_F32)
    hb = hb + w1[128:129, :] + b1[...]
    h[...] = hb
    grow = r * 128 + lax.broadcasted_iota(_I32, (128, 1), 0)
    m = grow < _NU
    hm = jnp.where(m, hb, 0.0)
    acc[0:1, :] += jnp.sum(hm, axis=0, keepdims=True)
    acc[1:2, :] += jnp.sum(jnp.where(m, hb * hb, 0.0), axis=0, keepdims=True)

    @pl.when(r == _PU // 128 - 1)
    def _():
        stats[...] = acc[...]


def _tc_mm(u0p, u1, u2, W1, b1r, lw):
    blk = lambda: pl.BlockSpec((128, _D), lambda r: (r, 0))
    return pl.pallas_call(
        _tc_mm_body,
        grid=(_PU // 128,),
        in_specs=[blk(), blk(), blk(),
                  pl.BlockSpec((129, _D), lambda r: (0, 0)),
                  pl.BlockSpec((1, _D), lambda r: (0, 0)),
                  pl.BlockSpec(memory_space=pltpu.SMEM)],
        out_specs=[blk(), pl.BlockSpec((2, _D), lambda r: (0, 0))],
        out_shape=[jax.ShapeDtypeStruct((_PU, _D), _F32),
                   jax.ShapeDtypeStruct((2, _D), _F32)],
        scratch_shapes=[pltpu.VMEM((2, _D), _F32)],
    )(u0p, u1, u2, W1, b1r, lw)


def _tc_norm_body(h, stats, g, b, o):
    mean = stats[0:1, :] * (1.0 / _NU)
    var = stats[1:2, :] * (1.0 / _NU) - mean * mean
    y = (h[...] - mean) * lax.rsqrt(var + 1e-5) * g[...] + b[...]
    o[...] = jnp.where(y > 0.0, y, jnp.exp(y) - 1.0)


def _tc_norm(h, stats, gr, br):
    blk = lambda: pl.BlockSpec((128, _D), lambda r: (r, 0))
    return pl.pallas_call(
        _tc_norm_body,
        grid=(_PU // 128,),
        in_specs=[blk(),
                  pl.BlockSpec((2, _D), lambda r: (0, 0)),
                  pl.BlockSpec((1, _D), lambda r: (0, 0)),
                  pl.BlockSpec((1, _D), lambda r: (0, 0))],
        out_specs=blk(),
        out_shape=jax.ShapeDtypeStruct((_PU, _D), _F32),
    )(h, stats, gr, br)


def kernel(edge_src, edge_dst, cell_feature, gene_feature, W1, b1, gamma,
           beta, layer_weights):
    pad = _EP - _E
    src2d = jnp.concatenate(
        [edge_src.astype(_I32), jnp.full((pad,), _NU, _I32)]).reshape(_ER, 128)
    dst2d = jnp.concatenate(
        [edge_dst.astype(_I32), jnp.full((pad,), _NI, _I32)]).reshape(_ER, 128)
    u0p = jnp.pad(cell_feature, ((0, _PU - _NU), (0, 0)))
    i0p = jnp.pad(gene_feature, ((0, _PI - _NI), (0, 0)))

    degu, degi = _k0a(src2d, dst2d)
    cu = _tc_c(degu)
    ci = _tc_c(degi)
    up = _k0c(u0p, i0p, cu, ci)
    upq, ipq = up[:4], up[4:]

    i1q, jpq = _klayer(False, True, src2d, dst2d, upq, ci)
    u1q, vpq = _klayer(True, True, src2d, dst2d, ipq, cu)

    i2q, _unused = _klayer(False, False, src2d, dst2d, vpq, ci)
    u2q, _unused2 = _klayer(True, False, src2d, dst2d, jpq, cu)

    i1 = jnp.concatenate(i1q, axis=1)
    i2 = jnp.concatenate(i2q, axis=1)
    u1 = jnp.concatenate(u1q, axis=1)
    u2 = jnp.concatenate(u2q, axis=1)

    ifin = _tc_ifinal(i0p, i1, i2, layer_weights)
    h, stats = _tc_mm(u0p, u1, u2, W1, b1.reshape(1, _D), layer_weights)
    uemb = _tc_norm(h, stats, gamma.reshape(1, _D), beta.reshape(1, _D))

    sc2d = _kdec(src2d, dst2d, uemb, ifin)
    return sc2d.reshape(-1)[:_E]


# speedup vs baseline: 2.9604x; 1.0542x over previous
import functools

import jax
import jax.numpy as jnp
from jax import lax
from jax.experimental import pallas as pl
from jax.experimental.pallas import tpu as pltpu
from jax.experimental.pallas import tpu_sc as plsc

_E = 600000
_ER = 4864
_EP = _ER * 128
_D = 128
_NU, _NI = 50000, 10000
_PU, _PI = 51200, 10240
_CPT = _ER // 16
_DCPT = _ER // 32
_RTU = _PU // (16 * 128)
_RTI = _PI // (16 * 128)

_F32 = jnp.float32
_I32 = jnp.int32

_SC_PARAMS = pltpu.CompilerParams(needs_layout_passes=False,
                                  use_tc_tiling_on_sc=False)


@functools.lru_cache(maxsize=None)
def _mesh():
    return plsc.VectorSubcoreMesh(core_axis_name="c", subcore_axis_name="s")


def _splat(ref, j):
    return plsc.load_gather(ref, [jnp.full((16,), j, _I32)])


def _fill(ref, val, rows, width):
    v = jnp.full((16,), val, _F32)
    if rows is None:
        for b in range(width // 16):
            ref[pl.ds(16 * b, 16)] = v
        return

    def body(r, _):
        for b in range(width // 16):
            ref[r, pl.ds(16 * b, 16)] = v
        return _

    lax.fori_loop(0, rows, body, None)


def _k0a_body(src2d, dst2d, degu, degi, acc, idxb, onesv, tmpv, ssem):
    sid = lax.axis_index("s")
    cid = lax.axis_index("c")

    def side(idx2d, deg_out, rchunks):
        t = sid
        _fill(tmpv, 0.0, None, 128)
        zd = [pltpu.async_copy(
                  tmpv, acc.at[pl.ds((t * rchunks + r) * 128, 128)], ssem)
              for r in range(rchunks)]
        for zz in zd:
            zz.wait()
        plsc.subcore_barrier()

        pltpu.sync_copy(idx2d.at[pl.ds(t * _CPT, _CPT)], idxb)
        _fill(onesv, 1.0, None, 128)

        def sb(j8, _):
            sd = [pltpu.async_copy(onesv, acc.at[idxb.at[j8 * 8 + jj]],
                                   ssem, add=True)
                  for jj in range(8)]
            for ss in sd:
                ss.wait()
            return _

        lax.fori_loop(0, _CPT // 8, sb, None)
        plsc.subcore_barrier()

        def wb(r, _):
            R = (t * rchunks + r) * 128
            pltpu.sync_copy(acc.at[pl.ds(R, 128)], tmpv)
            pltpu.sync_copy(tmpv, deg_out.at[pl.ds(R, 128)])
            return _

        lax.fori_loop(0, rchunks, wb, None)

    @pl.when(cid == 0)
    def _():
        side(src2d, degu, _RTU)

    @pl.when(cid == 1)
    def _():
        side(dst2d, degi, _RTI)


def _k0a(src2d, dst2d):
    out_type = (
        jax.ShapeDtypeStruct((_PU,), _F32),
        jax.ShapeDtypeStruct((_PI,), _F32),
    )
    scratch = [
        pltpu.VMEM_SHARED((_PU,), _F32),
        pltpu.VMEM((_CPT, 128), _I32),
        pltpu.VMEM((128,), _F32),
        pltpu.VMEM((128,), _F32),
        pltpu.SemaphoreType.DMA,
    ]
    f = pl.kernel(_k0a_body, out_type=out_type, mesh=_mesh(),
                  scratch_types=scratch, compiler_params=_SC_PARAMS)
    return f(src2d, dst2d)


def _tc_c_body(d, o):
    dv = d[...]
    o[...] = jnp.where(dv > 0.0, lax.rsqrt(jnp.maximum(dv, 1.0)), 0.0)


def _tc_c(deg):
    n = deg.shape[0]
    d2 = deg.reshape(n // 128, 128)
    blk = pl.BlockSpec((n // 128, 128), lambda: (0, 0))
    c2 = pl.pallas_call(
        _tc_c_body,
        in_specs=[blk],
        out_specs=blk,
        out_shape=jax.ShapeDtypeStruct((n // 128, 128), _F32),
    )(d2)
    return c2.reshape(n)


def _k0c_body(u0p, i0p, cuv, civ, up0, up1, up2, up3, ip0, ip1, ip2, ip3,
              cbuf, fbuf, g0, g1, g2, g3, wsem):
    sid = lax.axis_index("s")
    cid = lax.axis_index("c")
    gbufs = [g0, g1, g2, g3]

    def side(feat, c_in, gouts, rchunks):
        t = sid

        def drain(_r):
            for gb, go in zip(gbufs, gouts):
                pltpu.make_async_copy(gb, go.at[pl.ds(0, 128)], wsem).wait()

        def wb(r, _):
            R = (t * rchunks + r) * 128

            @pl.when(r > 0)
            def _():
                drain(r)

            pltpu.sync_copy(c_in.at[pl.ds(R, 128)], cbuf)
            pltpu.sync_copy(feat.at[pl.ds(R, 128)], fbuf)

            def rb(r2, _):
                cs = _splat(cbuf, r2)
                for b in range(8):
                    v = fbuf[r2, pl.ds(16 * b, 16)] * cs
                    gbufs[b // 2][r2, pl.ds(16 * (b % 2), 16)] = v
                return _

            lax.fori_loop(0, 128, rb, None)
            for gb, go in zip(gbufs, gouts):
                pltpu.async_copy(gb, go.at[pl.ds(R, 128)], wsem)
            return _

        lax.fori_loop(0, rchunks, wb, None)
        drain(0)

    @pl.when(cid == 0)
    def _():
        side(u0p, cuv, [up0, up1, up2, up3], _RTU)

    @pl.when(cid == 1)
    def _():
        side(i0p, civ, [ip0, ip1, ip2, ip3], _RTI)


def _k0c(u0p, i0p, cu, ci):
    out_type = tuple([jax.ShapeDtypeStruct((_PU, 32), _F32)] * 4
                     + [jax.ShapeDtypeStruct((_PI, 32), _F32)] * 4)
    scratch = [
        pltpu.VMEM((128,), _F32),
        pltpu.VMEM((128, 128), _F32),
        pltpu.VMEM((128, 32), _F32),
        pltpu.VMEM((128, 32), _F32),
        pltpu.VMEM((128, 32), _F32),
        pltpu.VMEM((128, 32), _F32),
        pltpu.SemaphoreType.DMA,
    ]
    f = pl.kernel(_k0c_body, out_type=out_type, mesh=_mesh(),
                  scratch_types=scratch, compiler_params=_SC_PARAMS)
    return f(u0p, i0p, cu, ci)


def _make_gpass(t, acc, s0, d0, s1, d1, rows4, cbuf,
                gsem, ssem, wsem, isem, primed):
    abuf = rows4.at[0]
    prb = rows4.at[1]
    ngroups = _CPT // 8

    def gpass(tbl, natout, prout, gsrc, xsrc, cvec, rchunks):
        _fill(abuf, 0.0, 128, 32)

        def zb(r, _):
            pltpu.async_copy(abuf,
                             acc.at[pl.ds((t * rchunks + r) * 128, 128)],
                             wsem)
            return _

        lax.fori_loop(0, rchunks, zb, None)

        def zw(r, _):
            pltpu.make_async_copy(
                abuf, acc.at[pl.ds(t * rchunks * 128, 128)], wsem).wait()
            return _

        lax.fori_loop(0, rchunks, zw, None)
        plsc.subcore_barrier()

        pltpu.sync_copy(gsrc.at[pl.ds(t * _CPT, 8)], s0)
        pltpu.sync_copy(xsrc.at[pl.ds(t * _CPT, 8)], d0)

        def group(g, sa, da, sbn, dbn):
            @pl.when(g > 0)
            def _():
                pltpu.make_async_copy(gsrc.at[pl.ds(0, 8)], sa, isem).wait()
                pltpu.make_async_copy(xsrc.at[pl.ds(0, 8)], da, isem).wait()

            @pl.when(g + 1 < ngroups)
            def _():
                nb = t * _CPT + (g + 1) * 8
                pltpu.async_copy(gsrc.at[pl.ds(nb, 8)], sbn, isem)
                pltpu.async_copy(xsrc.at[pl.ds(nb, 8)], dbn, isem)

            sd = []
            for w in range(2):
                if w == 1:
                    for ss in sd:
                        ss.wait()
                    sd = []
                gd = [pltpu.async_copy(tbl.at[sa.at[4 * w + jj]],
                                       rows4.at[jj], gsem)
                      for jj in range(4)]
                for jj in range(4):
                    gd[jj].wait()
                    sd.append(pltpu.async_copy(
                        rows4.at[jj], acc.at[da.at[4 * w + jj]],
                        ssem, add=True))
            for ss in sd:
                ss.wait()

        def sb2(jp, _):
            group(2 * jp, s0, d0, s1, d1)
            group(2 * jp + 1, s1, d1, s0, d0)
            return _

        lax.fori_loop(0, ngroups // 2, sb2, None)
        plsc.subcore_barrier()

        def drain_w(_r):
            pltpu.make_async_copy(abuf, natout.at[pl.ds(0, 128)],
                                  wsem).wait()
            if primed:
                pltpu.make_async_copy(prb, prout.at[pl.ds(0, 128)],
                                      wsem).wait()

        def wb(r, _):
            R = (t * rchunks + r) * 128

            @pl.when(r > 0)
            def _():
                drain_w(r)

            pltpu.sync_copy(acc.at[pl.ds(R, 128)], abuf)
            pltpu.sync_copy(cvec.at[pl.ds(R, 128)], cbuf)

            def rb(r2, _):
                cs = _splat(cbuf, r2)
                cs2 = cs * cs
                for b in range(2):
                    v = abuf[r2, pl.ds(16 * b, 16)]
                    if primed:
                        prb[r2, pl.ds(16 * b, 16)] = v * cs2
                    abuf[r2, pl.ds(16 * b, 16)] = v * cs
                return _

            lax.fori_loop(0, 128, rb, None)
            pltpu.async_copy(abuf, natout.at[pl.ds(R, 128)], wsem)
            if primed:
                pltpu.async_copy(prb, prout.at[pl.ds(R, 128)], wsem)
            return _

        lax.fori_loop(0, rchunks, wb, None)
        drain_w(0)
        plsc.subcore_barrier()

    return gpass


def _klayer2_body(primed, src2d, dst2d,
                  u0, u1, u2, u3, g0, g1, g2, g3, cu, ci, *refs):
    gnat = refs[0:4]
    cnat = refs[4:8]
    if primed:
        gpr = refs[8:12]
        cpr = refs[12:16]
        scr = refs[16:]
    else:
        gpr = cpr = (None,) * 4
        scr = refs[8:]
    acc, s0, d0, s1, d1, rows4, cbuf, gsem, ssem, wsem, isem = scr
    utbl = [u0, u1, u2, u3]
    itbl = [g0, g1, g2, g3]
    sid = lax.axis_index("s")
    cid = lax.axis_index("c")
    gpass = _make_gpass(sid, acc, s0, d0, s1, d1, rows4, cbuf,
                        gsem, ssem, wsem, isem, primed)

    @pl.when(cid == 0)
    def _():
        for g in range(4):
            gpass(utbl[g], gnat[g], gpr[g], src2d, dst2d, ci, _RTI)

    @pl.when(cid == 1)
    def _():
        for g in range(4):
            gpass(itbl[g], cnat[g], cpr[g], dst2d, src2d, cu, _RTU)


def _klayer2(primed, src2d, dst2d, utbls, itbls, cu, ci):
    out_type = ([jax.ShapeDtypeStruct((_PI, 32), _F32)] * 4
                + [jax.ShapeDtypeStruct((_PU, 32), _F32)] * 4)
    if primed:
        out_type += ([jax.ShapeDtypeStruct((_PI, 32), _F32)] * 4
                     + [jax.ShapeDtypeStruct((_PU, 32), _F32)] * 4)
    scratch = [
        pltpu.VMEM_SHARED((_PU, 32), _F32),
        pltpu.VMEM((8, 128), _I32),
        pltpu.VMEM((8, 128), _I32),
        pltpu.VMEM((8, 128), _I32),
        pltpu.VMEM((8, 128), _I32),
        pltpu.VMEM((4, 128, 32), _F32),
        pltpu.VMEM((128,), _F32),
        pltpu.SemaphoreType.DMA,
        pltpu.SemaphoreType.DMA,
        pltpu.SemaphoreType.DMA,
        pltpu.SemaphoreType.DMA,
    ]
    body = functools.partial(_klayer2_body, primed)
    f = pl.kernel(body, out_type=tuple(out_type), mesh=_mesh(),
                  scratch_types=scratch, compiler_params=_SC_PARAMS)
    outs = f(src2d, dst2d, *utbls, *itbls, cu, ci)
    return outs[0:4], outs[4:8], outs[8:12], outs[12:16]


def _kdec_body(src2d, dst2d, uemb, ifin, out2d, sidx, didx, ur2, ir2, sbuf,
               gsem):
    sid = lax.axis_index("s")
    cid = lax.axis_index("c")
    wid = sid * 2 + cid

    pltpu.sync_copy(src2d.at[pl.ds(wid * _DCPT, _DCPT)], sidx)
    pltpu.sync_copy(dst2d.at[pl.ds(wid * _DCPT, _DCPT)], didx)
    lane0 = lax.iota(_I32, 16) == 0

    def fire(k, bb):
        pltpu.async_copy(uemb.at[sidx.at[k]], ur2.at[bb], gsem)
        pltpu.async_copy(ifin.at[didx.at[k]], ir2.at[bb], gsem)

    def drain(bb):
        pltpu.make_async_copy(uemb.at[sidx.at[0]], ur2.at[bb], gsem).wait()
        pltpu.make_async_copy(ifin.at[didx.at[0]], ir2.at[bb], gsem).wait()

    def compute(k, bb):
        ur = ur2.at[bb]
        ir = ir2.at[bb]

        def eb(e, _):
            a = ur[e, pl.ds(0, 16)] * ir[e, pl.ds(0, 16)]
            for b in range(1, 8):
                a = a + ur[e, pl.ds(16 * b, 16)] * ir[e, pl.ds(16 * b, 16)]
            s = jnp.sum(a)
            plsc.store_scatter(sbuf, [jnp.full((16,), e, _I32)],
                               jnp.full((16,), s, _F32), mask=lane0)
            return _

        lax.fori_loop(0, 128, eb, None)
        for b in range(8):
            x = sbuf[pl.ds(16 * b, 16)]
            sbuf[pl.ds(16 * b, 16)] = 1.0 / (1.0 + jnp.exp(-x))
        pltpu.sync_copy(sbuf, out2d.at[wid * _DCPT + k])

    npairs = _DCPT // 2
    fire(0, 0)

    def db(j2, _):
        k0 = 2 * j2
        fire(k0 + 1, 1)
        drain(0)
        compute(k0, 0)

        @pl.when(j2 + 1 < npairs)
        def _():
            fire(k0 + 2, 0)

        drain(1)
        compute(k0 + 1, 1)
        return _

    lax.fori_loop(0, npairs, db, None)


def _kdec(src2d, dst2d, uemb, ifin):
    scratch = [
        pltpu.VMEM((_DCPT, 128), _I32),
        pltpu.VMEM((_DCPT, 128), _I32),
        pltpu.VMEM((2, 128, 128), _F32),
        pltpu.VMEM((2, 128, 128), _F32),
        pltpu.VMEM((128,), _F32),
        pltpu.SemaphoreType.DMA,
    ]
    f = pl.kernel(_kdec_body,
                  out_type=jax.ShapeDtypeStruct((_ER, 128), _F32),
                  mesh=_mesh(), scratch_types=scratch,
                  compiler_params=_SC_PARAMS)
    return f(src2d, dst2d, uemb, ifin)


def _tc_ifinal_body(i0, i1, i2, w, o):
    o[...] = w[0] * i0[...] + w[1] * i1[...] + w[2] * i2[...]


def _tc_ifinal(i0p, i1, i2, lw):
    blk = lambda: pl.BlockSpec((128, _D), lambda r: (r, 0))
    return pl.pallas_call(
        _tc_ifinal_body,
        grid=(_PI // 128,),
        in_specs=[blk(), blk(), blk(),
                  pl.BlockSpec(memory_space=pltpu.SMEM)],
        out_specs=blk(),
        out_shape=jax.ShapeDtypeStruct((_PI, _D), _F32),
    )(i0p, i1, i2, lw)


def _tc_mm_body(u0, u1, u2, w1, b1, w, h, stats, acc):
    r = pl.program_id(0)

    @pl.when(r == 0)
    def _():
        acc[...] = jnp.zeros_like(acc)

    ufin = w[0] * u0[...] + w[1] * u1[...] + w[2] * u2[...]
    hb = jnp.dot(ufin, w1[0:128, :], preferred_element_type=_F32)
    hb = hb + w1[128:129, :] + b1[...]
    h[...] = hb
    grow = r * 128 + lax.broadcasted_iota(_I32, (128, 1), 0)
    m = grow < _NU
    hm = jnp.where(m, hb, 0.0)
    acc[0:1, :] += jnp.sum(hm, axis=0, keepdims=True)
    acc[1:2, :] += jnp.sum(jnp.where(m, hb * hb, 0.0), axis=0, keepdims=True)

    @pl.when(r == _PU // 128 - 1)
    def _():
        stats[...] = acc[...]


def _tc_mm(u0p, u1, u2, W1, b1r, lw):
    blk = lambda: pl.BlockSpec((128, _D), lambda r: (r, 0))
    return pl.pallas_call(
        _tc_mm_body,
        grid=(_PU // 128,),
        in_specs=[blk(), blk(), blk(),
                  pl.BlockSpec((129, _D), lambda r: (0, 0)),
                  pl.BlockSpec((1, _D), lambda r: (0, 0)),
                  pl.BlockSpec(memory_space=pltpu.SMEM)],
        out_specs=[blk(), pl.BlockSpec((2, _D), lambda r: (0, 0))],
        out_shape=[jax.ShapeDtypeStruct((_PU, _D), _F32),
                   jax.ShapeDtypeStruct((2, _D), _F32)],
        scratch_shapes=[pltpu.VMEM((2, _D), _F32)],
    )(u0p, u1, u2, W1, b1r, lw)


def _tc_norm_body(h, stats, g, b, o):
    mean = stats[0:1, :] * (1.0 / _NU)
    var = stats[1:2, :] * (1.0 / _NU) - mean * mean
    y = (h[...] - mean) * lax.rsqrt(var + 1e-5) * g[...] + b[...]
    o[...] = jnp.where(y > 0.0, y, jnp.exp(y) - 1.0)


def _tc_norm(h, stats, gr, br):
    blk = lambda: pl.BlockSpec((128, _D), lambda r: (r, 0))
    return pl.pallas_call(
        _tc_norm_body,
        grid=(_PU // 128,),
        in_specs=[blk(),
                  pl.BlockSpec((2, _D), lambda r: (0, 0)),
                  pl.BlockSpec((1, _D), lambda r: (0, 0)),
                  pl.BlockSpec((1, _D), lambda r: (0, 0))],
        out_specs=blk(),
        out_shape=jax.ShapeDtypeStruct((_PU, _D), _F32),
    )(h, stats, gr, br)


def kernel(edge_src, edge_dst, cell_feature, gene_feature, W1, b1, gamma,
           beta, layer_weights):
    pad = _EP - _E
    src2d = jnp.concatenate(
        [edge_src.astype(_I32), jnp.full((pad,), _NU, _I32)]).reshape(_ER, 128)
    dst2d = jnp.concatenate(
        [edge_dst.astype(_I32), jnp.full((pad,), _NI, _I32)]).reshape(_ER, 128)
    u0p = jnp.pad(cell_feature, ((0, _PU - _NU), (0, 0)))
    i0p = jnp.pad(gene_feature, ((0, _PI - _NI), (0, 0)))

    degu, degi = _k0a(src2d, dst2d)
    cu = _tc_c(degu)
    ci = _tc_c(degi)
    up = _k0c(u0p, i0p, cu, ci)
    upq, ipq = up[:4], up[4:]

    i1q, u1q, ip1q, up1q = _klayer2(True, src2d, dst2d, upq, ipq, cu, ci)

    i2q, u2q, _unused, _unused2 = _klayer2(False, src2d, dst2d,
                                           up1q, ip1q, cu, ci)

    i1 = jnp.concatenate(i1q, axis=1)
    i2 = jnp.concatenate(i2q, axis=1)
    u1 = jnp.concatenate(u1q, axis=1)
    u2 = jnp.concatenate(u2q, axis=1)

    ifin = _tc_ifinal(i0p, i1, i2, layer_weights)
    h, stats = _tc_mm(u0p, u1, u2, W1, b1.reshape(1, _D), layer_weights)
    uemb = _tc_norm(h, stats, gamma.reshape(1, _D), beta.reshape(1, _D))

    sc2d = _kdec(src2d, dst2d, uemb, ifin)
    return sc2d.reshape(-1)[:_E]
